# baseline (device time: 50653 ns/iter reference)
import jax
import jax.numpy as jnp
from jax import lax
from jax.experimental import pallas as pl
from jax.experimental.pallas import tpu as pltpu

N_DEV = 4
SQ = 1024
SKV_SH = 1024
HQ = 8
DH = 128
D = HQ * DH
CHUNK = SQ // N_DEV
BLK = 64
SCALE = 0.08838834764831843


def _gather_group(ref, g):
    return jnp.concatenate(
        [ref[pl.ds(BLK * g + CHUNK * t, BLK), :] for t in range(4)], axis=0)


def _body(x_ref, wq_ref, k_ref, v_ref, wo_ref, out_ref,
          ctx_acc, l_acc, rs_ctx, rs_l, ag_buf,
          rs_ctx_ssem, rs_ctx_rsem, rs_l_ssem, rs_l_rsem,
          ag_ssem, ag_rsem):
    my_pos = lax.axis_index("i")

    barrier_sem = pltpu.get_barrier_semaphore()
    for dlt in (1, 2, 3):
        nbr = lax.rem(my_pos + dlt, N_DEV)
        pl.semaphore_signal(barrier_sem, inc=1, device_id=(nbr,),
                            device_id_type=pl.DeviceIdType.MESH)

    wq_bf = wq_ref[...].astype(jnp.bfloat16)

    ji = lax.broadcasted_iota(jnp.int32, (HQ * CHUNK, HQ), 0)
    hh = lax.broadcasted_iota(jnp.int32, (HQ * CHUNK, HQ), 1)
    ones_blk = (ji // CHUNK == hh).astype(jnp.bfloat16)

    barrier_waited = False
    rs_sends = []
    for dlt in (1, 2, 3, 0):
        tgt = lax.rem(my_pos + dlt, N_DEV)
        t_off = tgt * CHUNK
        xg = _gather_group(x_ref, tgt).astype(jnp.bfloat16)
        kg = jnp.concatenate(
            [k_ref[pl.ds(BLK * tgt + CHUNK * t, BLK), :, :]
             for t in range(4)], axis=0).astype(jnp.bfloat16)
        vg = jnp.concatenate(
            [v_ref[pl.ds(BLK * tgt + CHUNK * t, BLK), :, :]
             for t in range(4)], axis=0).astype(jnp.bfloat16)
        qg = jnp.dot(xg, wq_bf,
                     preferred_element_type=jnp.float32).astype(jnp.bfloat16)

        cs, ws = [], []
        for h in range(HQ):
            c0, c1 = h * DH, (h + 1) * DH
            s = lax.dot_general(qg[:, c0:c1], kg[:, h, :],
                                (((1,), (1,)), ((), ())),
                                preferred_element_type=jnp.float32) * SCALE
            w = jnp.exp(s).astype(jnp.bfloat16)
            ws.append(w)
            cs.append(jnp.dot(w, vg[:, h, :],
                              preferred_element_type=jnp.float32))
        ctx_acc[pl.ds(t_off, CHUNK), :] = jnp.concatenate(
            cs, axis=1).astype(jnp.bfloat16)
        l_acc[pl.ds(t_off, CHUNK), :] = jnp.dot(
            jnp.concatenate(ws, axis=1), ones_blk,
            preferred_element_type=jnp.float32)

        if dlt != 0:
            if not barrier_waited:
                pl.semaphore_wait(barrier_sem, N_DEV - 1)
                barrier_waited = True
            k_slot = dlt - 1
            ctx_rdma = pltpu.make_async_remote_copy(
                src_ref=ctx_acc.at[pl.ds(t_off, CHUNK)],
                dst_ref=rs_ctx.at[k_slot],
                send_sem=rs_ctx_ssem.at[k_slot],
                recv_sem=rs_ctx_rsem.at[k_slot],
                device_id=(tgt,), device_id_type=pl.DeviceIdType.MESH)
            l_rdma = pltpu.make_async_remote_copy(
                src_ref=l_acc.at[pl.ds(t_off, CHUNK)],
                dst_ref=rs_l.at[k_slot],
                send_sem=rs_l_ssem.at[k_slot],
                recv_sem=rs_l_rsem.at[k_slot],
                device_id=(tgt,), device_id_type=pl.DeviceIdType.MESH)
            ctx_rdma.start()
            l_rdma.start()
            rs_sends.append((ctx_rdma, l_rdma))

    for ctx_rdma, l_rdma in rs_sends:
        ctx_rdma.wait_recv()
        l_rdma.wait_recv()

    off = my_pos * CHUNK
    l_tot = (l_acc[pl.ds(off, CHUNK), :] + rs_l[0] + rs_l[1] + rs_l[2])
    inv_l = 1.0 / l_tot
    c_own = ctx_acc[pl.ds(off, CHUNK), :].astype(jnp.float32)
    c_sum = (c_own + rs_ctx[0].astype(jnp.float32)
             + rs_ctx[1].astype(jnp.float32) + rs_ctx[2].astype(jnp.float32))
    cols = []
    for h in range(HQ):
        c0, c1 = h * DH, (h + 1) * DH
        cols.append(c_sum[:, c0:c1] * inv_l[:, h:h + 1])
    ctx_chunk = jnp.concatenate(cols, axis=1)

    out_chunk = jnp.dot(ctx_chunk.astype(jnp.bfloat16),
                        wo_ref[...].astype(jnp.bfloat16),
                        preferred_element_type=jnp.float32)
    ag_buf[pl.ds(off, CHUNK), :] = out_chunk.astype(jnp.bfloat16)

    ag_sends = []
    for dlt in (1, 2, 3):
        tgt = lax.rem(my_pos + dlt, N_DEV)
        k_slot = dlt - 1
        ag_rdma = pltpu.make_async_remote_copy(
            src_ref=ag_buf.at[pl.ds(off, CHUNK)],
            dst_ref=ag_buf.at[pl.ds(off, CHUNK)],
            send_sem=ag_ssem.at[k_slot], recv_sem=ag_rsem.at[k_slot],
            device_id=(tgt,), device_id_type=pl.DeviceIdType.MESH)
        ag_rdma.start()
        ag_sends.append(ag_rdma)

    def _unpermute(g):
        for t in range(4):
            out_ref[pl.ds(BLK * g + CHUNK * t, BLK), :] = (
                ag_buf[pl.ds(CHUNK * g + BLK * t, BLK), :]
                .astype(jnp.float32))

    _unpermute(my_pos)

    for ctx_rdma, l_rdma in rs_sends:
        ctx_rdma.wait_send()
        l_rdma.wait_send()
    for k_slot, ag_rdma in enumerate(ag_sends):
        ag_rdma.wait_recv()
        _unpermute(lax.rem(my_pos + (N_DEV - 1 - k_slot), N_DEV))
    for ag_rdma in ag_sends:
        ag_rdma.wait_send()


def kernel(x, Wq, K_ext, V_ext, Wo):
    x2 = x.reshape(SQ, D)
    k2 = K_ext.reshape(SKV_SH, HQ, DH)
    v2 = V_ext.reshape(SKV_SH, HQ, DH)

    out = pl.pallas_call(
        _body,
        out_shape=jax.ShapeDtypeStruct((SQ, D), jnp.float32),
        in_specs=[pl.BlockSpec(memory_space=pltpu.VMEM)] * 5,
        out_specs=pl.BlockSpec(memory_space=pltpu.VMEM),
        scratch_shapes=[
            pltpu.VMEM((SQ, D), jnp.bfloat16),
            pltpu.VMEM((SQ, HQ), jnp.float32),
            pltpu.VMEM((3, CHUNK, D), jnp.bfloat16),
            pltpu.VMEM((3, CHUNK, HQ), jnp.float32),
            pltpu.VMEM((SQ, D), jnp.bfloat16),
            pltpu.SemaphoreType.DMA((3,)),
            pltpu.SemaphoreType.DMA((3,)),
            pltpu.SemaphoreType.DMA((3,)),
            pltpu.SemaphoreType.DMA((3,)),
            pltpu.SemaphoreType.DMA((3,)),
            pltpu.SemaphoreType.DMA((3,)),
        ],
        compiler_params=pltpu.CompilerParams(
            collective_id=0, vmem_limit_bytes=100 * 1024 * 1024),
    )(x2, Wq, k2, v2, Wo)
    return out.reshape(1, SQ, D)


# device time: 50178 ns/iter; 1.0095x vs baseline; 1.0095x over previous
import jax
import jax.numpy as jnp
from jax import lax
from jax.experimental import pallas as pl
from jax.experimental.pallas import tpu as pltpu

N_DEV = 4
SQ = 1024
SKV_SH = 1024
HQ = 8
DH = 128
D = HQ * DH
CHUNK = SQ // N_DEV
BLK = 64
SCALE = 0.08838834764831843


def _gather_group(ref, g):
    return jnp.concatenate(
        [ref[pl.ds(BLK * g + CHUNK * t, BLK), :] for t in range(4)], axis=0)


def _body(x_ref, wq_ref, k_ref, v_ref, wo_ref, out_ref,
          ctx_acc, l_acc, rs_ctx, rs_l, ag_buf,
          rs_ctx_ssem, rs_ctx_rsem, rs_l_ssem, rs_l_rsem,
          ag_ssem, ag_rsem):
    my_pos = lax.axis_index("i")

    barrier_sem = pltpu.get_barrier_semaphore()
    for dlt in (1, 2, 3):
        nbr = lax.rem(my_pos + dlt, N_DEV)
        pl.semaphore_signal(barrier_sem, inc=1, device_id=(nbr,),
                            device_id_type=pl.DeviceIdType.MESH)

    wq_bf = wq_ref[...].astype(jnp.bfloat16)

    barrier_waited = False
    rs_sends = []
    for dlt in (1, 2, 3, 0):
        tgt = lax.rem(my_pos + dlt, N_DEV)
        t_off = tgt * CHUNK
        xg = _gather_group(x_ref, tgt).astype(jnp.bfloat16)
        kg = jnp.concatenate(
            [k_ref[pl.ds(BLK * tgt + CHUNK * t, BLK), :, :]
             for t in range(4)], axis=0).astype(jnp.bfloat16)
        vg = jnp.concatenate(
            [v_ref[pl.ds(BLK * tgt + CHUNK * t, BLK), :, :]
             for t in range(4)], axis=0).astype(jnp.bfloat16)
        qg = jnp.dot(xg, wq_bf,
                     preferred_element_type=jnp.float32).astype(jnp.bfloat16)

        cs, ls = [], []
        for h in range(HQ):
            c0, c1 = h * DH, (h + 1) * DH
            s = lax.dot_general(qg[:, c0:c1], kg[:, h, :],
                                (((1,), (1,)), ((), ())),
                                preferred_element_type=jnp.float32) * SCALE
            w = jnp.exp(s)
            ls.append(jnp.sum(w, axis=1, keepdims=True))
            cs.append(jnp.dot(w.astype(jnp.bfloat16), vg[:, h, :],
                              preferred_element_type=jnp.float32))
        ctx_acc[pl.ds(t_off, CHUNK), :] = jnp.concatenate(
            cs, axis=1).astype(jnp.bfloat16)
        l_acc[pl.ds(t_off, CHUNK), :] = jnp.concatenate(ls, axis=1)

        if dlt != 0:
            if not barrier_waited:
                pl.semaphore_wait(barrier_sem, N_DEV - 1)
                barrier_waited = True
            k_slot = dlt - 1
            ctx_rdma = pltpu.make_async_remote_copy(
                src_ref=ctx_acc.at[pl.ds(t_off, CHUNK)],
                dst_ref=rs_ctx.at[k_slot],
                send_sem=rs_ctx_ssem.at[k_slot],
                recv_sem=rs_ctx_rsem.at[k_slot],
                device_id=(tgt,), device_id_type=pl.DeviceIdType.MESH)
            l_rdma = pltpu.make_async_remote_copy(
                src_ref=l_acc.at[pl.ds(t_off, CHUNK)],
                dst_ref=rs_l.at[k_slot],
                send_sem=rs_l_ssem.at[k_slot],
                recv_sem=rs_l_rsem.at[k_slot],
                device_id=(tgt,), device_id_type=pl.DeviceIdType.MESH)
            ctx_rdma.start()
            l_rdma.start()
            rs_sends.append((ctx_rdma, l_rdma))

    for ctx_rdma, l_rdma in rs_sends:
        ctx_rdma.wait_recv()
        l_rdma.wait_recv()

    off = my_pos * CHUNK
    l_tot = (l_acc[pl.ds(off, CHUNK), :] + rs_l[0] + rs_l[1] + rs_l[2])
    inv_l = 1.0 / l_tot
    c_own = ctx_acc[pl.ds(off, CHUNK), :].astype(jnp.float32)
    c_sum = (c_own + rs_ctx[0].astype(jnp.float32)
             + rs_ctx[1].astype(jnp.float32) + rs_ctx[2].astype(jnp.float32))
    cols = []
    for h in range(HQ):
        c0, c1 = h * DH, (h + 1) * DH
        cols.append(c_sum[:, c0:c1] * inv_l[:, h:h + 1])
    ctx_chunk = jnp.concatenate(cols, axis=1)

    out_chunk = jnp.dot(ctx_chunk.astype(jnp.bfloat16),
                        wo_ref[...].astype(jnp.bfloat16),
                        preferred_element_type=jnp.float32)
    ag_buf[pl.ds(off, CHUNK), :] = out_chunk.astype(jnp.bfloat16)

    ag_sends = []
    for dlt in (1, 2, 3):
        tgt = lax.rem(my_pos + dlt, N_DEV)
        k_slot = dlt - 1
        ag_rdma = pltpu.make_async_remote_copy(
            src_ref=ag_buf.at[pl.ds(off, CHUNK)],
            dst_ref=ag_buf.at[pl.ds(off, CHUNK)],
            send_sem=ag_ssem.at[k_slot], recv_sem=ag_rsem.at[k_slot],
            device_id=(tgt,), device_id_type=pl.DeviceIdType.MESH)
        ag_rdma.start()
        ag_sends.append(ag_rdma)

    def _unpermute(g):
        for t in range(4):
            out_ref[pl.ds(BLK * g + CHUNK * t, BLK), :] = (
                ag_buf[pl.ds(CHUNK * g + BLK * t, BLK), :]
                .astype(jnp.float32))

    _unpermute(my_pos)

    for ctx_rdma, l_rdma in rs_sends:
        ctx_rdma.wait_send()
        l_rdma.wait_send()
    for k_slot, ag_rdma in enumerate(ag_sends):
        ag_rdma.wait_recv()
        _unpermute(lax.rem(my_pos + (N_DEV - 1 - k_slot), N_DEV))
    for ag_rdma in ag_sends:
        ag_rdma.wait_send()


def kernel(x, Wq, K_ext, V_ext, Wo):
    x2 = x.reshape(SQ, D)
    k2 = K_ext.reshape(SKV_SH, HQ, DH)
    v2 = V_ext.reshape(SKV_SH, HQ, DH)

    out = pl.pallas_call(
        _body,
        out_shape=jax.ShapeDtypeStruct((SQ, D), jnp.float32),
        in_specs=[pl.BlockSpec(memory_space=pltpu.VMEM)] * 5,
        out_specs=pl.BlockSpec(memory_space=pltpu.VMEM),
        scratch_shapes=[
            pltpu.VMEM((SQ, D), jnp.bfloat16),
            pltpu.VMEM((SQ, HQ), jnp.float32),
            pltpu.VMEM((3, CHUNK, D), jnp.bfloat16),
            pltpu.VMEM((3, CHUNK, HQ), jnp.float32),
            pltpu.VMEM((SQ, D), jnp.bfloat16),
            pltpu.SemaphoreType.DMA((3,)),
            pltpu.SemaphoreType.DMA((3,)),
            pltpu.SemaphoreType.DMA((3,)),
            pltpu.SemaphoreType.DMA((3,)),
            pltpu.SemaphoreType.DMA((3,)),
            pltpu.SemaphoreType.DMA((3,)),
        ],
        compiler_params=pltpu.CompilerParams(
            collective_id=0, vmem_limit_bytes=100 * 1024 * 1024),
    )(x2, Wq, k2, v2, Wo)
    return out.reshape(1, SQ, D)
